# baseline (speedup 1.0000x reference)
import jax
import jax.numpy as jnp
from jax.experimental import pallas as pl
from jax.experimental.pallas import tpu as pltpu

_USER = 8192
_ITEM = 8192
_DIM = 64
_TM = 512


def _layer_body(xt_ref, gammat_ref, betat_ref, adjl_ref, adjr_ref,
                et_ref,
                bnt_ref, bni_ref, iacct_ref):
    i = pl.program_id(0)
    ni = pl.num_programs(0)

    @pl.when(i == 0)
    def _init():
        xt = xt_ref[...]
        mean = jnp.mean(xt, axis=1, keepdims=True)
        var = jnp.mean((xt - mean) ** 2, axis=1, keepdims=True)
        s = gammat_ref[...] * jax.lax.rsqrt(var + 1e-5)
        t = betat_ref[...] - mean * s
        bnt = (xt * s + t).astype(jnp.bfloat16)
        bnt_ref[...] = bnt
        bni_ref[...] = jnp.transpose(bnt[:, _USER:])
        iacct_ref[...] = jnp.zeros_like(iacct_ref)

    al = adjl_ref[...].astype(jnp.bfloat16)
    ar = adjr_ref[...].astype(jnp.bfloat16)
    half = _ITEM // 2

    ug = (jax.lax.dot_general(
        al, bni_ref[:half, :],
        dimension_numbers=(((1,), (0,)), ((), ())),
        preferred_element_type=jnp.float32)
        + jax.lax.dot_general(
        ar, bni_ref[half:, :],
        dimension_numbers=(((1,), (0,)), ((), ())),
        preferred_element_type=jnp.float32))
    et_ref[:, pl.ds(i * _TM, _TM)] = jnp.transpose(ug)

    bnt_blk = bnt_ref[:, pl.ds(i * _TM, _TM)]
    iacct_ref[:, :half] += jax.lax.dot_general(
        bnt_blk, al,
        dimension_numbers=(((1,), (0,)), ((), ())),
        preferred_element_type=jnp.float32)
    iacct_ref[:, half:] += jax.lax.dot_general(
        bnt_blk, ar,
        dimension_numbers=(((1,), (0,)), ((), ())),
        preferred_element_type=jnp.float32)

    @pl.when(i == ni - 1)
    def _fin():
        et_ref[:, _USER:] = iacct_ref[...]


def _fused_layer(adj, xt, gammat, betat):
    n_blk = _USER // _TM
    return pl.pallas_call(
        _layer_body,
        grid=(n_blk,),
        in_specs=[
            pl.BlockSpec((_DIM, _USER + _ITEM), lambda i: (0, 0)),
            pl.BlockSpec((_DIM, 1), lambda i: (0, 0)),
            pl.BlockSpec((_DIM, 1), lambda i: (0, 0)),
            pl.BlockSpec((_TM, _ITEM // 2), lambda i: (i, 0)),
            pl.BlockSpec((_TM, _ITEM // 2), lambda i: (i, 1)),
        ],
        out_specs=pl.BlockSpec((_DIM, _USER + _ITEM), lambda i: (0, 0)),
        out_shape=jax.ShapeDtypeStruct((_DIM, _USER + _ITEM), jnp.float32),
        scratch_shapes=[
            pltpu.VMEM((_DIM, _USER + _ITEM), jnp.bfloat16),
            pltpu.VMEM((_ITEM, _DIM), jnp.bfloat16),
            pltpu.VMEM((_DIM, _ITEM), jnp.float32),
        ],
        compiler_params=pltpu.CompilerParams(
            dimension_semantics=("arbitrary",)),
    )(xt, gammat, betat, adj, adj)


def kernel(adj, embeds, bn_gamma, bn_beta):
    x = embeds
    xt = jnp.transpose(embeds)
    lats = [embeds]
    gcn_lats = [embeds]
    for layer in range(2):
        gt = bn_gamma[layer][:, None]
        bt = bn_beta[layer][:, None]
        et = _fused_layer(adj, xt, gt, bt)
        e = jnp.transpose(et)
        gcn_lats.append(e)
        x = x + e
        xt = xt + et
        lats.append(x)
    return (jnp.stack(lats), jnp.stack(gcn_lats))


# speedup vs baseline: 1.0257x; 1.0257x over previous
import jax
import jax.numpy as jnp
from jax.experimental import pallas as pl
from jax.experimental.pallas import tpu as pltpu

_USER = 8192
_ITEM = 8192
_DIM = 64
_TM = 512


def _layer_body(xt_ref, gammat_ref, betat_ref, adj_ref,
                et_ref,
                bnt_ref, bni_ref, iacct_ref):
    i = pl.program_id(0)
    ni = pl.num_programs(0)

    @pl.when(i == 0)
    def _init():
        xt = xt_ref[...]
        mean = jnp.mean(xt, axis=1, keepdims=True)
        var = jnp.mean((xt - mean) ** 2, axis=1, keepdims=True)
        s = gammat_ref[...] * jax.lax.rsqrt(var + 1e-5)
        t = betat_ref[...] - mean * s
        bnt = (xt * s + t).astype(jnp.bfloat16)
        bnt_ref[...] = bnt
        bni_ref[...] = jnp.transpose(bnt[:, _USER:])
        iacct_ref[...] = jnp.zeros_like(iacct_ref)

    a = adj_ref[...].astype(jnp.bfloat16)

    ug = jax.lax.dot_general(
        a, bni_ref[...],
        dimension_numbers=(((1,), (0,)), ((), ())),
        preferred_element_type=jnp.float32)
    et_ref[:, pl.ds(i * _TM, _TM)] = jnp.transpose(ug)

    iacct_ref[...] += jax.lax.dot_general(
        bnt_ref[:, pl.ds(i * _TM, _TM)], a,
        dimension_numbers=(((1,), (0,)), ((), ())),
        preferred_element_type=jnp.float32)

    @pl.when(i == ni - 1)
    def _fin():
        et_ref[:, _USER:] = iacct_ref[...]


def _fused_layer(adj, xt, gammat, betat):
    n_blk = _USER // _TM
    return pl.pallas_call(
        _layer_body,
        grid=(n_blk,),
        in_specs=[
            pl.BlockSpec((_DIM, _USER + _ITEM), lambda i: (0, 0)),
            pl.BlockSpec((_DIM, 1), lambda i: (0, 0)),
            pl.BlockSpec((_DIM, 1), lambda i: (0, 0)),
            pl.BlockSpec((_TM, _ITEM), lambda i: (i, 0)),
        ],
        out_specs=pl.BlockSpec((_DIM, _USER + _ITEM), lambda i: (0, 0)),
        out_shape=jax.ShapeDtypeStruct((_DIM, _USER + _ITEM), jnp.float32),
        scratch_shapes=[
            pltpu.VMEM((_DIM, _USER + _ITEM), jnp.bfloat16),
            pltpu.VMEM((_ITEM, _DIM), jnp.bfloat16),
            pltpu.VMEM((_DIM, _ITEM), jnp.float32),
        ],
        compiler_params=pltpu.CompilerParams(
            dimension_semantics=("arbitrary",)),
    )(xt, gammat, betat, adj)


def kernel(adj, embeds, bn_gamma, bn_beta):
    x = embeds
    xt = jnp.transpose(embeds)
    lats = [embeds]
    gcn_lats = [embeds]
    for layer in range(2):
        gt = bn_gamma[layer][:, None]
        bt = bn_beta[layer][:, None]
        et = _fused_layer(adj, xt, gt, bt)
        e = jnp.transpose(et)
        gcn_lats.append(e)
        x = x + e
        xt = xt + et
        lats.append(x)
    return (jnp.stack(lats), jnp.stack(gcn_lats))
